# Initial kernel scaffold; baseline (speedup 1.0000x reference)
import functools

import jax
import jax.numpy as jnp
from jax import lax
from jax.experimental import pallas as pl
from jax.experimental.pallas import tpu as pltpu
from jax.experimental.pallas import tpu_sc as plsc

N = 10000
D = 128
E = 320000
NC = 2
NS = 16
NW = NC * NS
CH = 128
K = -(-E // (NW * CH))
EP = NW * CH * K
ROWS_PER_TILE = 640
NP = NS * ROWS_PER_TILE


def _fill(ref, rows, cols, value):
    v = jnp.full((16,), value, jnp.float32)

    def row(i, _):
        def col(j, _):
            ref[i, pl.ds(j * 16, 16)] = v
            return 0
        return lax.fori_loop(0, cols // 16, col, 0)

    lax.fori_loop(0, rows, row, 0)


def _make_sc_agg(with_deg: bool):
    mesh = plsc.VectorSubcoreMesh(core_axis_name="c", subcore_axis_name="s")
    out_type = [jax.ShapeDtypeStruct((NC, NP, D), jnp.float32)]
    scratch = [
        pltpu.VMEM((K, CH), jnp.int32),
        pltpu.VMEM((K, CH), jnp.int32),
        pltpu.VMEM((CH, D), jnp.float32),
        pltpu.VMEM((CH, D), jnp.float32),
        pltpu.VMEM_SHARED((NP, D), jnp.float32),
        pltpu.SemaphoreType.DMA,
    ]
    if with_deg:
        out_type.append(jax.ShapeDtypeStruct((NC, NP, 16), jnp.float32))
        scratch += [
            pltpu.VMEM((CH, 16), jnp.float32),
            pltpu.VMEM((CH, 16), jnp.float32),
            pltpu.VMEM_SHARED((NP, 16), jnp.float32),
        ]

    def body(z_hbm, src_hbm, dst_hbm, out_hbm, *rest):
        if with_deg:
            (deg_hbm, idx_s, idx_d, rows, zrows, acc_sh, sem,
             ones16, zdeg, deg_sh) = rest
        else:
            (idx_s, idx_d, rows, zrows, acc_sh, sem) = rest
        cid = lax.axis_index("c")
        sid = lax.axis_index("s")
        w = cid * NS + sid
        base = sid * ROWS_PER_TILE

        _fill(zrows, CH, D, 0.0)
        if with_deg:
            _fill(ones16, CH, 16, 1.0)
            _fill(zdeg, CH, 16, 0.0)

        def zacc(t, _):
            pltpu.sync_copy(zrows, acc_sh.at[pl.ds(base + t * CH, CH)])
            if with_deg:
                pltpu.sync_copy(zdeg, deg_sh.at[pl.ds(base + t * CH, CH)])
            return 0
        lax.fori_loop(0, ROWS_PER_TILE // CH, zacc, 0)

        pltpu.sync_copy(src_hbm.at[w], idx_s)
        pltpu.sync_copy(dst_hbm.at[w], idx_d)
        plsc.subcore_barrier()

        def step(j, _):
            pltpu.async_copy(z_hbm.at[idx_s.at[j]], rows, sem).wait()
            pltpu.sync_copy(rows, acc_sh.at[idx_d.at[j]], add=True)
            if with_deg:
                pltpu.sync_copy(ones16, deg_sh.at[idx_d.at[j]], add=True)
            return 0
        lax.fori_loop(0, K, step, 0)

        plsc.subcore_barrier()

        pltpu.sync_copy(acc_sh.at[pl.ds(base, ROWS_PER_TILE)],
                        out_hbm.at[cid].at[pl.ds(base, ROWS_PER_TILE)])
        if with_deg:
            pltpu.sync_copy(deg_sh.at[pl.ds(base, ROWS_PER_TILE)],
                            deg_hbm.at[cid].at[pl.ds(base, ROWS_PER_TILE)])

    return pl.kernel(body, out_type=tuple(out_type), mesh=mesh,
                     scratch_types=scratch)


_sc_agg_deg = _make_sc_agg(with_deg=True)
_sc_agg = _make_sc_agg(with_deg=False)


def _dense_body(z_ref, p_ref, g_ref, w_ref, b_ref, o_ref):
    deg = jnp.maximum(g_ref[0, :, 0:1] + g_ref[1, :, 0:1], 1.0)
    zn = (p_ref[0] + p_ref[1]) / deg
    x = (jnp.dot(z_ref[...], w_ref[:D], preferred_element_type=jnp.float32)
         + jnp.dot(zn, w_ref[D:], preferred_element_type=jnp.float32)
         + b_ref[...])
    h = jax.nn.sigmoid(x)
    nrm = jnp.sqrt(jnp.sum(h * h, axis=1, keepdims=True))
    o_ref[...] = h / jnp.maximum(nrm, 1e-12)


_R = 640
_tc_dense = pl.pallas_call(
    _dense_body,
    grid=(NP // _R,),
    in_specs=[
        pl.BlockSpec((_R, D), lambda i: (i, 0)),
        pl.BlockSpec((NC, _R, D), lambda i: (0, i, 0)),
        pl.BlockSpec((NC, _R, 16), lambda i: (0, i, 0)),
        pl.BlockSpec((2 * D, D), lambda i: (0, 0)),
        pl.BlockSpec((1, D), lambda i: (0, 0)),
    ],
    out_specs=pl.BlockSpec((_R, D), lambda i: (i, 0)),
    out_shape=jax.ShapeDtypeStruct((NP, D), jnp.float32),
)


@jax.jit
def kernel(Z, edge_index, W0, b0, W1, b1):
    Zp = jnp.zeros((NP, D), jnp.float32).at[:N].set(Z)
    pad = EP - E
    src = jnp.concatenate([edge_index[0], jnp.zeros((pad,), jnp.int32)])
    dst = jnp.concatenate([edge_index[1], jnp.full((pad,), N, jnp.int32)])
    src3 = src.reshape(NW, K, CH)
    dst3 = dst.reshape(NW, K, CH)

    agg1, degp = _sc_agg_deg(Zp, src3, dst3)
    Z1p = _tc_dense(Zp, agg1, degp, W0, b0.reshape(1, D))
    (agg2,) = _sc_agg(Z1p, src3, dst3)
    Z2p = _tc_dense(Z1p, agg2, degp, W1, b1.reshape(1, D))
    return Z2p[:N]


# speedup vs baseline: 6.0220x; 6.0220x over previous
import functools

import jax
import jax.numpy as jnp
from jax import lax
from jax.experimental import pallas as pl
from jax.experimental.pallas import tpu as pltpu
from jax.experimental.pallas import tpu_sc as plsc

N = 10000
D = 128
E = 320000
DH = D // 2
NC = 2
NS = 16
CH = 128
K = -(-E // (NS * CH))
KH = (K + 1) // 2
EP = NS * CH * K
ROWS_PER_TILE = 640
NP = NS * ROWS_PER_TILE


def _fill(ref, rows, cols, value):
    v = jnp.full((16,), value, jnp.float32)

    def row(i, _):
        def col(j, _):
            ref[i, pl.ds(j * 16, 16)] = v
            return 0
        return lax.fori_loop(0, cols // 16, col, 0)

    lax.fori_loop(0, rows, row, 0)


def _make_sc_agg(with_deg: bool):
    mesh = plsc.VectorSubcoreMesh(core_axis_name="c", subcore_axis_name="s",
                                  num_cores=NC, num_subcores=NS)
    out_type = [jax.ShapeDtypeStruct((NC, NP, DH), jnp.float32)]
    scratch = [
        pltpu.VMEM((K, CH), jnp.int32),
        pltpu.VMEM((K, CH), jnp.int32),
        pltpu.VMEM((CH, DH), jnp.float32),
        pltpu.VMEM((CH, DH), jnp.float32),
        pltpu.VMEM_SHARED((NP, DH), jnp.float32),
        pltpu.SemaphoreType.DMA,
    ]
    if with_deg:
        out_type.append(jax.ShapeDtypeStruct((NC, NP, 16), jnp.float32))
        scratch += [
            pltpu.VMEM((CH, 16), jnp.float32),
            pltpu.VMEM((CH, 16), jnp.float32),
            pltpu.VMEM_SHARED((NP, 16), jnp.float32),
        ]

    def body(zl_hbm, zr_hbm, src_hbm, dst_hbm, out_hbm, *rest):
        if with_deg:
            (deg_hbm, idx_s, idx_d, rows, zrows, acc_sh, sem,
             ones16, zdeg, deg_sh) = rest
        else:
            (idx_s, idx_d, rows, zrows, acc_sh, sem) = rest
        cid = lax.axis_index("c")
        sid = lax.axis_index("s")
        base = sid * ROWS_PER_TILE

        _fill(zrows, CH, DH, 0.0)
        if with_deg:
            _fill(ones16, CH, 16, 1.0)
            _fill(zdeg, CH, 16, 0.0)

        def zacc(t, _):
            pltpu.sync_copy(zrows, acc_sh.at[pl.ds(base + t * CH, CH)])
            if with_deg:
                pltpu.sync_copy(zdeg, deg_sh.at[pl.ds(base + t * CH, CH)])
            return 0
        lax.fori_loop(0, ROWS_PER_TILE // CH, zacc, 0)

        pltpu.sync_copy(src_hbm.at[sid], idx_s)
        pltpu.sync_copy(dst_hbm.at[sid], idx_d)
        plsc.subcore_barrier()

        def make_step(z_hbm, deg_lo, deg_hi):
            def step(j, _):
                pltpu.async_copy(z_hbm.at[idx_s.at[j]], rows, sem).wait()
                pltpu.sync_copy(rows, acc_sh.at[idx_d.at[j]], add=True)
                if with_deg:
                    @pl.when(jnp.logical_and(j >= deg_lo, j < deg_hi))
                    def _():
                        pltpu.sync_copy(ones16, deg_sh.at[idx_d.at[j]],
                                        add=True)
                return 0
            return step

        @pl.when(cid == 0)
        def _():
            lax.fori_loop(0, K, make_step(zl_hbm, 0, KH), 0)

        @pl.when(cid == 1)
        def _():
            lax.fori_loop(0, K, make_step(zr_hbm, KH, K), 0)

        plsc.subcore_barrier()

        pltpu.sync_copy(acc_sh.at[pl.ds(base, ROWS_PER_TILE)],
                        out_hbm.at[cid].at[pl.ds(base, ROWS_PER_TILE)])
        if with_deg:
            pltpu.sync_copy(deg_sh.at[pl.ds(base, ROWS_PER_TILE)],
                            deg_hbm.at[cid].at[pl.ds(base, ROWS_PER_TILE)])

    return pl.kernel(
        body, out_type=tuple(out_type), mesh=mesh, scratch_types=scratch,
        compiler_params=pltpu.CompilerParams(use_tc_tiling_on_sc=False))


_make_sc_agg = functools.cache(_make_sc_agg)


def _dense_body(z_ref, p_ref, g_ref, w_ref, b_ref, o_ref):
    deg = jnp.maximum(g_ref[0, :, 0:1] + g_ref[1, :, 0:1], 1.0)
    zn = jnp.concatenate((p_ref[0], p_ref[1]), axis=1) / deg
    x = (jnp.dot(z_ref[...], w_ref[:D], preferred_element_type=jnp.float32)
         + jnp.dot(zn, w_ref[D:], preferred_element_type=jnp.float32)
         + b_ref[...])
    h = jax.nn.sigmoid(x)
    nrm = jnp.sqrt(jnp.sum(h * h, axis=1, keepdims=True))
    o_ref[...] = h / jnp.maximum(nrm, 1e-12)


_R = 640
_tc_dense = pl.pallas_call(
    _dense_body,
    grid=(NP // _R,),
    in_specs=[
        pl.BlockSpec((_R, D), lambda i: (i, 0)),
        pl.BlockSpec((NC, _R, DH), lambda i: (0, i, 0)),
        pl.BlockSpec((NC, _R, 16), lambda i: (0, i, 0)),
        pl.BlockSpec((2 * D, D), lambda i: (0, 0)),
        pl.BlockSpec((1, D), lambda i: (0, 0)),
    ],
    out_specs=pl.BlockSpec((_R, D), lambda i: (i, 0)),
    out_shape=jax.ShapeDtypeStruct((NP, D), jnp.float32),
)


def kernel(Z, edge_index, W0, b0, W1, b1):
    Zp = jnp.zeros((NP, D), jnp.float32).at[:N].set(Z)
    pad = EP - E
    src = jnp.concatenate([edge_index[0], jnp.zeros((pad,), jnp.int32)])
    dst = jnp.concatenate([edge_index[1], jnp.full((pad,), N, jnp.int32)])
    src3 = src.reshape(NS, K, CH)
    dst3 = dst.reshape(NS, K, CH)

    agg1, degp = _make_sc_agg(True)(Zp[:, :DH], Zp[:, DH:], src3, dst3)
    Z1p = _tc_dense(Zp, agg1, degp, W0, b0.reshape(1, D))
    (agg2,) = _make_sc_agg(False)(Z1p[:, :DH], Z1p[:, DH:], src3, dst3)
    Z2p = _tc_dense(Z1p, agg2, degp, W1, b1.reshape(1, D))
    return Z2p[:N]


# speedup vs baseline: 7.3793x; 1.2254x over previous
import functools

import jax
import jax.numpy as jnp
from jax import lax
from jax.experimental import pallas as pl
from jax.experimental.pallas import tpu as pltpu
from jax.experimental.pallas import tpu_sc as plsc

N = 10000
D = 128
E = 320000
DH = D // 2
NC = 2
NS = 16
CH = 128
K = -(-E // (NS * CH))
KH = (K + 1) // 2
EP = NS * CH * K
ROWS_PER_TILE = 640
NP = NS * ROWS_PER_TILE


def _fill(ref, rows, cols, value):
    v = jnp.full((16,), value, jnp.float32)

    def row(i, _):
        def col(j, _):
            ref[i, pl.ds(j * 16, 16)] = v
            return 0
        return lax.fori_loop(0, cols // 16, col, 0)

    lax.fori_loop(0, rows, row, 0)


def _make_sc_agg(with_deg: bool):
    mesh = plsc.VectorSubcoreMesh(core_axis_name="c", subcore_axis_name="s",
                                  num_cores=NC, num_subcores=NS)
    out_type = [jax.ShapeDtypeStruct((NC, NP, DH), jnp.float32)]
    scratch = [
        pltpu.VMEM((K, CH), jnp.int32),
        pltpu.VMEM((K, CH), jnp.int32),
        pltpu.VMEM((2 * CH, DH), jnp.float32),
        pltpu.VMEM((CH, DH), jnp.float32),
        pltpu.VMEM_SHARED((NP, DH), jnp.float32),
        pltpu.SemaphoreType.DMA,
    ]
    if with_deg:
        out_type.append(jax.ShapeDtypeStruct((NC, NP, 16), jnp.float32))
        scratch += [
            pltpu.VMEM((CH, 16), jnp.float32),
            pltpu.VMEM((CH, 16), jnp.float32),
            pltpu.VMEM_SHARED((NP, 16), jnp.float32),
        ]

    def body(zl_hbm, zr_hbm, src_hbm, dst_hbm, out_hbm, *rest):
        if with_deg:
            (deg_hbm, idx_s, idx_d, rows, zrows, acc_sh, sem,
             ones16, zdeg, deg_sh) = rest
        else:
            (idx_s, idx_d, rows, zrows, acc_sh, sem) = rest
        cid = lax.axis_index("c")
        sid = lax.axis_index("s")
        base = sid * ROWS_PER_TILE

        _fill(zrows, CH, DH, 0.0)
        if with_deg:
            _fill(ones16, CH, 16, 1.0)
            _fill(zdeg, CH, 16, 0.0)

        def zacc(t, _):
            pltpu.sync_copy(zrows, acc_sh.at[pl.ds(base + t * CH, CH)])
            if with_deg:
                pltpu.sync_copy(zdeg, deg_sh.at[pl.ds(base + t * CH, CH)])
            return 0
        lax.fori_loop(0, ROWS_PER_TILE // CH, zacc, 0)

        pltpu.sync_copy(src_hbm.at[sid], idx_s)
        pltpu.sync_copy(dst_hbm.at[sid], idx_d)
        plsc.subcore_barrier()

        def make_loop(z_hbm, deg_lo, deg_hi):
            def run():
                for b in range(1):
                    pltpu.async_copy(z_hbm.at[idx_s.at[b]],
                                     rows.at[pl.ds(b * CH, CH)], sem)

                def step(j, _):
                    slot = lax.rem(j, 2)
                    buf = rows.at[pl.ds(slot * CH, CH)]
                    pltpu.make_async_copy(z_hbm.at[idx_s.at[j]], buf,
                                          sem).wait()
                    @pl.when(j + 1 < K)
                    def _():
                        nslot = lax.rem(j + 1, 2)
                        pltpu.async_copy(
                            z_hbm.at[idx_s.at[j + 1]],
                            rows.at[pl.ds(nslot * CH, CH)], sem)
                    pltpu.sync_copy(buf, acc_sh.at[idx_d.at[j]], add=True)
                    if with_deg:
                        @pl.when(jnp.logical_and(j >= deg_lo, j < deg_hi))
                        def _():
                            pltpu.sync_copy(ones16, deg_sh.at[idx_d.at[j]],
                                            add=True)
                    return 0
                lax.fori_loop(0, K, step, 0)
            return run

        @pl.when(cid == 0)
        def _():
            make_loop(zl_hbm, 0, KH)()

        @pl.when(cid == 1)
        def _():
            make_loop(zr_hbm, KH, K)()

        plsc.subcore_barrier()

        pltpu.sync_copy(acc_sh.at[pl.ds(base, ROWS_PER_TILE)],
                        out_hbm.at[cid].at[pl.ds(base, ROWS_PER_TILE)])
        if with_deg:
            pltpu.sync_copy(deg_sh.at[pl.ds(base, ROWS_PER_TILE)],
                            deg_hbm.at[cid].at[pl.ds(base, ROWS_PER_TILE)])

    return pl.kernel(
        body, out_type=tuple(out_type), mesh=mesh, scratch_types=scratch,
        compiler_params=pltpu.CompilerParams(use_tc_tiling_on_sc=False))


_make_sc_agg = functools.cache(_make_sc_agg)


def _dense_body(z_ref, p_ref, g_ref, w_ref, b_ref, o_ref):
    deg = jnp.maximum(g_ref[0, :, 0:1] + g_ref[1, :, 0:1], 1.0)
    zn = jnp.concatenate((p_ref[0], p_ref[1]), axis=1) / deg
    x = (jnp.dot(z_ref[...], w_ref[:D], preferred_element_type=jnp.float32)
         + jnp.dot(zn, w_ref[D:], preferred_element_type=jnp.float32)
         + b_ref[...])
    h = jax.nn.sigmoid(x)
    nrm = jnp.sqrt(jnp.sum(h * h, axis=1, keepdims=True))
    o_ref[...] = h / jnp.maximum(nrm, 1e-12)


_R = 640
_tc_dense = pl.pallas_call(
    _dense_body,
    grid=(NP // _R,),
    in_specs=[
        pl.BlockSpec((_R, D), lambda i: (i, 0)),
        pl.BlockSpec((NC, _R, DH), lambda i: (0, i, 0)),
        pl.BlockSpec((NC, _R, 16), lambda i: (0, i, 0)),
        pl.BlockSpec((2 * D, D), lambda i: (0, 0)),
        pl.BlockSpec((1, D), lambda i: (0, 0)),
    ],
    out_specs=pl.BlockSpec((_R, D), lambda i: (i, 0)),
    out_shape=jax.ShapeDtypeStruct((NP, D), jnp.float32),
)


def kernel(Z, edge_index, W0, b0, W1, b1):
    Zp = jnp.zeros((NP, D), jnp.float32).at[:N].set(Z)
    pad = EP - E
    src = jnp.concatenate([edge_index[0], jnp.zeros((pad,), jnp.int32)])
    dst = jnp.concatenate([edge_index[1], jnp.full((pad,), N, jnp.int32)])
    src3 = src.reshape(NS, K, CH)
    dst3 = dst.reshape(NS, K, CH)

    agg1, degp = _make_sc_agg(True)(Zp[:, :DH], Zp[:, DH:], src3, dst3)
    Z1p = _tc_dense(Zp, agg1, degp, W0, b0.reshape(1, D))
    (agg2,) = _make_sc_agg(False)(Z1p[:, :DH], Z1p[:, DH:], src3, dst3)
    Z2p = _tc_dense(Z1p, agg2, degp, W1, b1.reshape(1, D))
    return Z2p[:N]


# speedup vs baseline: 9.5911x; 1.2997x over previous
import functools

import jax
import jax.numpy as jnp
from jax import lax
from jax.experimental import pallas as pl
from jax.experimental.pallas import tpu as pltpu
from jax.experimental.pallas import tpu_sc as plsc

N = 10000
D = 128
E = 320000
DH = D // 2
NC = 2
NS = 16
CH = 128
K = -(-E // (NS * CH))
KH = (K + 1) // 2
EP = NS * CH * K
ROWS_PER_TILE = 640
NP = NS * ROWS_PER_TILE


def _fill(ref, rows, cols, value):
    v = jnp.full((16,), value, jnp.float32)

    def row(i, _):
        def col(j, _):
            ref[i, pl.ds(j * 16, 16)] = v
            return 0
        return lax.fori_loop(0, cols // 16, col, 0)

    lax.fori_loop(0, rows, row, 0)


def _make_sc_agg(with_deg: bool):
    mesh = plsc.VectorSubcoreMesh(core_axis_name="c", subcore_axis_name="s",
                                  num_cores=NC, num_subcores=NS)
    out_type = [jax.ShapeDtypeStruct((NC, NP, DH), jnp.float32)]
    scratch = [
        pltpu.VMEM((K, CH), jnp.int32),
        pltpu.VMEM((K, CH), jnp.int32),
        pltpu.VMEM((3 * CH, DH), jnp.float32),
        pltpu.VMEM((CH, DH), jnp.float32),
        pltpu.VMEM_SHARED((NP, DH), jnp.float32),
        pltpu.SemaphoreType.DMA,
        pltpu.SemaphoreType.DMA,
    ]
    if with_deg:
        out_type.append(jax.ShapeDtypeStruct((NC, NP, 16), jnp.float32))
        scratch += [
            pltpu.VMEM((CH, 16), jnp.float32),
            pltpu.VMEM((CH, 16), jnp.float32),
            pltpu.VMEM_SHARED((NP, 16), jnp.float32),
        ]

    def body(zl_hbm, zr_hbm, src_hbm, dst_hbm, out_hbm, *rest):
        if with_deg:
            (deg_hbm, idx_s, idx_d, rows, zrows, acc_sh, sem, sem_s,
             ones16, zdeg, deg_sh) = rest
        else:
            (idx_s, idx_d, rows, zrows, acc_sh, sem, sem_s) = rest
        cid = lax.axis_index("c")
        sid = lax.axis_index("s")
        base = sid * ROWS_PER_TILE

        _fill(zrows, CH, DH, 0.0)
        if with_deg:
            _fill(ones16, CH, 16, 1.0)
            _fill(zdeg, CH, 16, 0.0)

        def zacc(t, _):
            pltpu.sync_copy(zrows, acc_sh.at[pl.ds(base + t * CH, CH)])
            if with_deg:
                pltpu.sync_copy(zdeg, deg_sh.at[pl.ds(base + t * CH, CH)])
            return 0
        lax.fori_loop(0, ROWS_PER_TILE // CH, zacc, 0)

        pltpu.sync_copy(src_hbm.at[sid], idx_s)
        pltpu.sync_copy(dst_hbm.at[sid], idx_d)
        plsc.subcore_barrier()

        def make_loop(z_hbm, deg_lo, deg_hi):
            def run():
                for b in range(2):
                    pltpu.async_copy(z_hbm.at[idx_s.at[b]],
                                     rows.at[pl.ds(b * CH, CH)], sem)

                def step(j, _):
                    slot = lax.rem(j, 3)
                    buf = rows.at[pl.ds(slot * CH, CH)]
                    pltpu.make_async_copy(z_hbm.at[idx_s.at[j]], buf,
                                          sem).wait()
                    @pl.when(j >= 1)
                    def _():
                        pslot = lax.rem(j + 2, 3)
                        pbuf = rows.at[pl.ds(pslot * CH, CH)]
                        pltpu.make_async_copy(
                            pbuf, acc_sh.at[idx_d.at[j - 1]], sem_s).wait()
                    @pl.when(j + 2 < K)
                    def _():
                        nslot = lax.rem(j + 2, 3)
                        pltpu.async_copy(
                            z_hbm.at[idx_s.at[j + 2]],
                            rows.at[pl.ds(nslot * CH, CH)], sem)
                    pltpu.async_copy(buf, acc_sh.at[idx_d.at[j]], sem_s,
                                     add=True)
                    if with_deg:
                        @pl.when(jnp.logical_and(j >= deg_lo, j < deg_hi))
                        def _():
                            pltpu.sync_copy(ones16, deg_sh.at[idx_d.at[j]],
                                            add=True)
                    return 0
                lax.fori_loop(0, K, step, 0)
                pltpu.make_async_copy(
                    rows.at[pl.ds(((K - 1) % 3) * CH, CH)],
                    acc_sh.at[idx_d.at[K - 1]], sem_s).wait()
            return run

        @pl.when(cid == 0)
        def _():
            make_loop(zl_hbm, 0, KH)()

        @pl.when(cid == 1)
        def _():
            make_loop(zr_hbm, KH, K)()

        plsc.subcore_barrier()

        pltpu.sync_copy(acc_sh.at[pl.ds(base, ROWS_PER_TILE)],
                        out_hbm.at[cid].at[pl.ds(base, ROWS_PER_TILE)])
        if with_deg:
            pltpu.sync_copy(deg_sh.at[pl.ds(base, ROWS_PER_TILE)],
                            deg_hbm.at[cid].at[pl.ds(base, ROWS_PER_TILE)])

    return pl.kernel(
        body, out_type=tuple(out_type), mesh=mesh, scratch_types=scratch,
        compiler_params=pltpu.CompilerParams(use_tc_tiling_on_sc=False))


_make_sc_agg = functools.cache(_make_sc_agg)


def _dense_body(z_ref, p_ref, g_ref, w_ref, b_ref, o_ref):
    deg = jnp.maximum(g_ref[0, :, 0:1] + g_ref[1, :, 0:1], 1.0)
    zn = jnp.concatenate((p_ref[0], p_ref[1]), axis=1) / deg
    x = (jnp.dot(z_ref[...], w_ref[:D], preferred_element_type=jnp.float32)
         + jnp.dot(zn, w_ref[D:], preferred_element_type=jnp.float32)
         + b_ref[...])
    h = jax.nn.sigmoid(x)
    nrm = jnp.sqrt(jnp.sum(h * h, axis=1, keepdims=True))
    o_ref[...] = h / jnp.maximum(nrm, 1e-12)


_R = 640
_tc_dense = pl.pallas_call(
    _dense_body,
    grid=(NP // _R,),
    in_specs=[
        pl.BlockSpec((_R, D), lambda i: (i, 0)),
        pl.BlockSpec((NC, _R, DH), lambda i: (0, i, 0)),
        pl.BlockSpec((NC, _R, 16), lambda i: (0, i, 0)),
        pl.BlockSpec((2 * D, D), lambda i: (0, 0)),
        pl.BlockSpec((1, D), lambda i: (0, 0)),
    ],
    out_specs=pl.BlockSpec((_R, D), lambda i: (i, 0)),
    out_shape=jax.ShapeDtypeStruct((NP, D), jnp.float32),
)


def kernel(Z, edge_index, W0, b0, W1, b1):
    Zp = jnp.zeros((NP, D), jnp.float32).at[:N].set(Z)
    pad = EP - E
    src = jnp.concatenate([edge_index[0], jnp.zeros((pad,), jnp.int32)])
    dst = jnp.concatenate([edge_index[1], jnp.full((pad,), N, jnp.int32)])
    src3 = src.reshape(NS, K, CH)
    dst3 = dst.reshape(NS, K, CH)

    agg1, degp = _make_sc_agg(True)(Zp[:, :DH], Zp[:, DH:], src3, dst3)
    Z1p = _tc_dense(Zp, agg1, degp, W0, b0.reshape(1, D))
    (agg2,) = _make_sc_agg(False)(Z1p[:, :DH], Z1p[:, DH:], src3, dst3)
    Z2p = _tc_dense(Z1p, agg2, degp, W1, b1.reshape(1, D))
    return Z2p[:N]
